# baseline (speedup 1.0000x reference)
import functools

import jax
import jax.numpy as jnp
from jax import lax
from jax.experimental import pallas as pl
from jax.experimental.pallas import tpu as pltpu
from jax.experimental.pallas import tpu_sc as plsc

D_MODEL = 64
SCALE = 8.0

_NC = 2
_NS = 16
_NW = _NC * _NS
_CHUNK = 128
_LANES = 16


@functools.lru_cache(maxsize=None)
def _make_kernel(B: int):
    assert B % (_NW * _CHUNK) == 0
    b_per_w = B // _NW
    n_chunks = b_per_w // _CHUNK
    mesh = plsc.VectorSubcoreMesh(core_axis_name="c", subcore_axis_name="s")

    @functools.partial(
        pl.kernel,
        mesh=mesh,
        out_type=jax.ShapeDtypeStruct((B, D_MODEL), jnp.float32),
        compiler_params=pltpu.CompilerParams(use_tc_tiling_on_sc=False),
        scratch_types=[
            pltpu.VMEM((n_chunks, _CHUNK), jnp.int32),
            pltpu.VMEM((_CHUNK, D_MODEL), jnp.float32),
            pltpu.SemaphoreType.DMA,
        ],
    )
    def k(idx_hbm, table_hbm, out_hbm, idx_v, buf, sem):
        wid = lax.axis_index("s") * _NC + lax.axis_index("c")
        base = wid * b_per_w
        pltpu.sync_copy(idx_hbm.at[wid], idx_v)

        def chunk_body(j, carry):
            pltpu.async_copy(table_hbm.at[idx_v.at[j]], buf, sem).wait()

            def row_body(r, c2):
                for c in range(D_MODEL // _LANES):
                    sl = pl.ds(c * _LANES, _LANES)
                    buf[r, sl] = buf[r, sl] * SCALE
                return c2

            lax.fori_loop(0, _CHUNK, row_body, 0)
            pltpu.sync_copy(buf, out_hbm.at[pl.ds(base + j * _CHUNK, _CHUNK)])
            return carry

        lax.fori_loop(0, n_chunks, chunk_body, 0)

    return k


def kernel(x, lut):
    B = x.size
    idx = x.reshape(_NW, B // (_NW * _CHUNK), _CHUNK).astype(jnp.int32)
    out = _make_kernel(B)(idx, lut)
    return out.reshape(x.shape + (D_MODEL,))


# speedup vs baseline: 1.2042x; 1.2042x over previous
import functools

import jax
import jax.numpy as jnp
from jax import lax
from jax.experimental import pallas as pl
from jax.experimental.pallas import tpu as pltpu
from jax.experimental.pallas import tpu_sc as plsc

D_MODEL = 64
SCALE = 8.0

_NC = 2
_NS = 16
_NW = _NC * _NS
_CHUNK = 128
_LANES = 16
_NBUF = 4
_DEPTH = 2


@functools.lru_cache(maxsize=None)
def _make_kernel(B: int):
    assert B % (_NW * _CHUNK) == 0
    b_per_w = B // _NW
    n_chunks = b_per_w // _CHUNK
    n_super = n_chunks // _NBUF
    assert n_chunks % _NBUF == 0 and n_super >= 2
    mesh = plsc.VectorSubcoreMesh(core_axis_name="c", subcore_axis_name="s")

    @functools.partial(
        pl.kernel,
        mesh=mesh,
        out_type=jax.ShapeDtypeStruct((B, D_MODEL), jnp.float32),
        compiler_params=pltpu.CompilerParams(use_tc_tiling_on_sc=False),
        scratch_types=(
            [pltpu.VMEM((n_chunks, _CHUNK), jnp.int32)]
            + [pltpu.VMEM((_CHUNK, D_MODEL), jnp.float32)] * _NBUF
            + [pltpu.SemaphoreType.DMA] * (2 * _NBUF)
        ),
    )
    def k(idx_hbm, table_hbm, out_hbm, idx_v, *rest):
        bufs = rest[:_NBUF]
        gsem = rest[_NBUF:2 * _NBUF]
        ssem = rest[2 * _NBUF:]
        wid = lax.axis_index("s") * _NC + lax.axis_index("c")
        base = wid * b_per_w
        pltpu.sync_copy(idx_hbm.at[wid], idx_v)

        def fire_gather(j, b):
            pltpu.async_copy(table_hbm.at[idx_v.at[j]], bufs[b], gsem[b])

        def wait_gather(b):
            pltpu.make_async_copy(table_hbm.at[idx_v.at[0]], bufs[b], gsem[b]).wait()

        def out_slice(j):
            return out_hbm.at[pl.ds(base + j * _CHUNK, _CHUNK)]

        def fire_store(j, b):
            pltpu.async_copy(bufs[b], out_slice(j), ssem[b])

        def wait_store(b):
            pltpu.make_async_copy(bufs[b], out_slice(0), ssem[b]).wait()

        def scale_buf(b):
            def row_body(r, c2):
                for c in range(D_MODEL // _LANES):
                    sl = pl.ds(c * _LANES, _LANES)
                    bufs[b][r, sl] = bufs[b][r, sl] * SCALE
                return c2

            lax.fori_loop(0, _CHUNK, row_body, 0)

        fire_gather(0, 0)
        fire_gather(1, 1)

        for b in range(_NBUF):
            if b >= _DEPTH:
                wait_store((b + _DEPTH) % _NBUF)
            fire_gather(b + _DEPTH, (b + _DEPTH) % _NBUF)
            wait_gather(b)
            scale_buf(b)
            fire_store(b, b)

        def super_body(g, carry):
            j0 = g * _NBUF
            for b in range(_NBUF):
                wait_store((b + _DEPTH) % _NBUF)
                fire_gather(j0 + b + _DEPTH, (b + _DEPTH) % _NBUF)
                wait_gather(b)
                scale_buf(b)
                fire_store(j0 + b, b)
            return carry

        lax.fori_loop(1, n_super - 1, super_body, 0)

        j0 = (n_super - 1) * _NBUF
        for b in range(_NBUF):
            wait_store((b + _DEPTH) % _NBUF)
            if b < _NBUF - _DEPTH:
                fire_gather(j0 + b + _DEPTH, (b + _DEPTH) % _NBUF)
            wait_gather(b)
            scale_buf(b)
            fire_store(j0 + b, b)

        for b in range(_NBUF - _DEPTH, _NBUF):
            wait_store(b)

    return k


def kernel(x, lut):
    B = x.size
    idx = x.reshape(_NW, B // (_NW * _CHUNK), _CHUNK).astype(jnp.int32)
    out = _make_kernel(B)(idx, lut)
    return out.reshape(x.shape + (D_MODEL,))
